# Initial kernel scaffold; baseline (speedup 1.0000x reference)
import jax
import jax.numpy as jnp
from jax.experimental import pallas as pl


def kernel(x, w_gate0, conv_w0, conv_b0, w_gate1, conv_w1, conv_b1, fc1_w, fc1_b, fc2_w, fc2_b):
    raise NotImplementedError("write your pallas kernel here")


# speedup vs baseline: 1.6969x; 1.6969x over previous
import jax
import jax.numpy as jnp
import numpy as np
from jax.experimental import pallas as pl
from jax.experimental.pallas import tpu as pltpu

E = 8
N = 16

_SEL0 = np.zeros((100, 64), np.float32)
for _h in range(8):
    for _w in range(8):
        _SEL0[_h * 10 + _w, _h * 8 + _w] = 1.0


def _shift(img, s):
    if s == 0:
        return img
    n, q = img.shape
    return jnp.concatenate([img[:, s:], jnp.zeros((n, s), img.dtype)], axis=1)


def _gates(logits):
    m = jnp.max(logits, axis=1, keepdims=True)
    e = jnp.exp(logits - m)
    p = e / jnp.sum(e, axis=1, keepdims=True)
    ecols = jax.lax.broadcasted_iota(jnp.int32, p.shape, 1)
    i1 = jnp.argmax(p, axis=1)[:, None]
    m1 = ecols == i1
    v1 = jnp.max(p, axis=1, keepdims=True)
    p2 = jnp.where(m1, -jnp.inf, p)
    i2 = jnp.argmax(p2, axis=1)[:, None]
    m2 = ecols == i2
    v2 = jnp.max(p2, axis=1, keepdims=True)
    d = v1 + v2 + 1e-6
    return m1 * (v1 / d) + m2 * (v2 / d)


def _moe_body(x_ref, wg0_ref, w0_ref, b0_ref, wg1_ref, w1_ref, b1_ref,
              sel0_ref, h1_ref):
    x = x_ref[...]
    g0 = _gates(x @ wg0_ref[...])
    S0 = [_shift(x, i * 10 + j) for i in range(3) for j in range(3)]
    w0 = w0_ref[...]
    b0 = b0_ref[...]
    h0q = jnp.zeros((N, 16, 100), jnp.float32)
    for ex in range(E):
        acc = jnp.zeros((N, 16, 100), jnp.float32)
        for k in range(9):
            wk = w0[ex * 16:(ex + 1) * 16, k]
            acc = acc + wk[None, :, None] * S0[k][:, None, :]
        r = jnp.maximum(acc + b0[0, ex * 16:(ex + 1) * 16][None, :, None], 0.0)
        h0q = h0q + g0[:, ex][:, None, None] * r
    sel0 = sel0_ref[...]
    img1 = jnp.concatenate(
        [jax.lax.dot(h0q[:, c, :], sel0) for c in range(16)], axis=1)

    g1 = _gates(img1 @ wg1_ref[...])
    S1 = [_shift(img1, i * 32 + j) for i in range(3) for j in range(3)]
    w1 = w1_ref[...]
    b1 = b1_ref[...]
    h1_ref[...] = jnp.zeros((N, 32, 1024), jnp.float32)
    for ex in range(E):
        acc = jnp.zeros((N, 32, 1024), jnp.float32)
        for k in range(9):
            wk = w1[ex * 32:(ex + 1) * 32, k]
            acc = acc + wk[None, :, None] * S1[k][:, None, :]
        r = jnp.maximum(acc + b1[0, ex * 32:(ex + 1) * 32][None, :, None], 0.0)
        h1_ref[...] += g1[:, ex][:, None, None] * r


_NB = 48
_CK = 460800 // _NB


def _head_body(flat_ref, w1_ref, b1_ref, w2_ref, b2_ref, out_ref, zacc_ref):
    i = pl.program_id(0)

    @pl.when(i == 0)
    def _():
        zacc_ref[...] = jnp.zeros_like(zacc_ref)

    zacc_ref[...] += jax.lax.dot_general(
        flat_ref[...], w1_ref[...], (((1,), (1,)), ((), ())),
        preferred_element_type=jnp.float32)

    @pl.when(i == _NB - 1)
    def _():
        z = jnp.maximum(zacc_ref[...] + b1_ref[...], 0.0)
        o = jax.lax.dot_general(
            z, w2_ref[...], (((1,), (1,)), ((), ())),
            preferred_element_type=jnp.float32)
        o = o + b2_ref[...]
        mx = jnp.max(o, axis=1, keepdims=True)
        l = o - mx
        out_ref[...] = l - jnp.log(jnp.sum(jnp.exp(l), axis=1, keepdims=True))


def kernel(x, w_gate0, conv_w0, conv_b0, w_gate1, conv_w1, conv_b1,
           fc1_w, fc1_b, fc2_w, fc2_b):
    w0 = conv_w0.reshape(E * 16, 9)
    b0 = conv_b0.reshape(1, E * 16)
    w1 = conv_w1.reshape(E * 32, 9)
    b1 = conv_b1.reshape(1, E * 32)

    h1_pad = pl.pallas_call(
        _moe_body,
        out_shape=jax.ShapeDtypeStruct((N, 32, 1024), jnp.float32),
    )(x, w_gate0, w0, b0, w_gate1, w1, b1, jnp.asarray(_SEL0))

    flat = h1_pad.reshape(N, 32, 32, 32)[:, :, :30, :30].reshape(1, 460800)

    out = pl.pallas_call(
        _head_body,
        grid=(_NB,),
        in_specs=[
            pl.BlockSpec((1, _CK), lambda i: (0, i)),
            pl.BlockSpec((128, _CK), lambda i: (0, i)),
            pl.BlockSpec((1, 128), lambda i: (0, 0)),
            pl.BlockSpec((28800, 128), lambda i: (0, 0)),
            pl.BlockSpec((1, 28800), lambda i: (0, 0)),
        ],
        out_specs=pl.BlockSpec((1, 28800), lambda i: (0, 0)),
        out_shape=jax.ShapeDtypeStruct((1, 28800), jnp.float32),
        scratch_shapes=[pltpu.VMEM((1, 128), jnp.float32)],
    )(flat, fc1_w, fc1_b.reshape(1, 128), fc2_w, fc2_b.reshape(1, 28800))

    return out


# speedup vs baseline: 1.7068x; 1.0058x over previous
import jax
import jax.numpy as jnp
import numpy as np
from jax.experimental import pallas as pl
from jax.experimental.pallas import tpu as pltpu

E = 8
N = 16

_SEL0 = np.zeros((100, 64), np.float32)
for _h in range(8):
    for _w in range(8):
        _SEL0[_h * 10 + _w, _h * 8 + _w] = 1.0


def _shift(img, s):
    if s == 0:
        return img
    n, q = img.shape
    return jnp.concatenate([img[:, s:], jnp.zeros((n, s), img.dtype)], axis=1)


def _gates(logits):
    m = jnp.max(logits, axis=1, keepdims=True)
    e = jnp.exp(logits - m)
    p = e / jnp.sum(e, axis=1, keepdims=True)
    ecols = jax.lax.broadcasted_iota(jnp.int32, p.shape, 1)
    i1 = jnp.argmax(p, axis=1)[:, None]
    m1 = ecols == i1
    v1 = jnp.max(p, axis=1, keepdims=True)
    p2 = jnp.where(m1, -jnp.inf, p)
    i2 = jnp.argmax(p2, axis=1)[:, None]
    m2 = ecols == i2
    v2 = jnp.max(p2, axis=1, keepdims=True)
    d = v1 + v2 + 1e-6
    return m1 * (v1 / d) + m2 * (v2 / d)


def _moe_body(x_ref, wg0_ref, w0_ref, b0_ref, wg1_ref, w1_ref, b1_ref,
              sel0_ref, h1_ref):
    x = x_ref[...]
    g0 = _gates(x @ wg0_ref[...])
    S0 = [_shift(x, i * 10 + j) for i in range(3) for j in range(3)]
    w0 = w0_ref[...]
    b0 = b0_ref[...]
    h0q = jnp.zeros((N, 16, 100), jnp.float32)
    for ex in range(E):
        acc = jnp.zeros((N, 16, 100), jnp.float32)
        for k in range(9):
            wk = w0[ex * 16:(ex + 1) * 16, k]
            acc = acc + wk[None, :, None] * S0[k][:, None, :]
        r = jnp.maximum(acc + b0[0, ex * 16:(ex + 1) * 16][None, :, None], 0.0)
        h0q = h0q + g0[:, ex][:, None, None] * r
    sel0 = sel0_ref[...]
    img1 = jnp.concatenate(
        [jax.lax.dot(h0q[:, c, :], sel0) for c in range(16)], axis=1)

    g1 = _gates(img1 @ wg1_ref[...])
    S1 = [_shift(img1, i * 32 + j) for i in range(3) for j in range(3)]
    w1 = w1_ref[...]
    b1 = b1_ref[...]
    h1_ref[...] = jnp.zeros((N, 32, 1024), jnp.float32)
    for ex in range(E):
        acc = jnp.zeros((N, 32, 1024), jnp.float32)
        for k in range(9):
            wk = w1[ex * 32:(ex + 1) * 32, k]
            acc = acc + wk[None, :, None] * S1[k][:, None, :]
        r = jnp.maximum(acc + b1[0, ex * 32:(ex + 1) * 32][None, :, None], 0.0)
        h1_ref[...] += g1[:, ex][:, None, None] * r


_NB = 16
_RK = 128 // _NB


def _fc1_body(flat_ref, w1_ref, z_ref):
    z_ref[...] = jax.lax.dot_general(
        w1_ref[...], flat_ref[...], (((1,), (1,)), ((), ())),
        preferred_element_type=jnp.float32)


def _fc2_body(z_ref, b1_ref, w2_ref, b2_ref, out_ref):
    z = jnp.maximum(z_ref[...] + b1_ref[...], 0.0)
    o = jax.lax.dot_general(
        z, w2_ref[...], (((1,), (1,)), ((), ())),
        preferred_element_type=jnp.float32)
    o = o + b2_ref[...]
    mx = jnp.max(o, axis=1, keepdims=True)
    l = o - mx
    out_ref[...] = l - jnp.log(jnp.sum(jnp.exp(l), axis=1, keepdims=True))


def kernel(x, w_gate0, conv_w0, conv_b0, w_gate1, conv_w1, conv_b1,
           fc1_w, fc1_b, fc2_w, fc2_b):
    w0 = conv_w0.reshape(E * 16, 9)
    b0 = conv_b0.reshape(1, E * 16)
    w1 = conv_w1.reshape(E * 32, 9)
    b1 = conv_b1.reshape(1, E * 32)

    h1_pad = pl.pallas_call(
        _moe_body,
        out_shape=jax.ShapeDtypeStruct((N, 32, 1024), jnp.float32),
    )(x, w_gate0, w0, b0, w_gate1, w1, b1, jnp.asarray(_SEL0))

    flat = h1_pad.reshape(N, 32, 32, 32)[:, :, :30, :30].reshape(1, 460800)

    zc = pl.pallas_call(
        _fc1_body,
        grid=(_NB,),
        in_specs=[
            pl.BlockSpec((1, 460800), lambda i: (0, 0)),
            pl.BlockSpec((_RK, 460800), lambda i: (i, 0)),
        ],
        out_specs=pl.BlockSpec((_RK, 1), lambda i: (i, 0)),
        out_shape=jax.ShapeDtypeStruct((128, 1), jnp.float32),
    )(flat, fc1_w)

    out = pl.pallas_call(
        _fc2_body,
        in_specs=[
            pl.BlockSpec((1, 128), lambda i: (0, 0)),
            pl.BlockSpec((1, 128), lambda i: (0, 0)),
            pl.BlockSpec((28800, 128), lambda i: (0, 0)),
            pl.BlockSpec((1, 28800), lambda i: (0, 0)),
        ],
        out_specs=pl.BlockSpec((1, 28800), lambda i: (0, 0)),
        out_shape=jax.ShapeDtypeStruct((1, 28800), jnp.float32),
        grid=(1,),
    )(zc.reshape(1, 128), fc1_b.reshape(1, 128), fc2_w, fc2_b.reshape(1, 28800))

    return out


# speedup vs baseline: 2.0198x; 1.1834x over previous
import jax
import jax.numpy as jnp
import numpy as np
from jax.experimental import pallas as pl
from jax.experimental.pallas import tpu as pltpu

E = 8
N = 16

_SEL0 = np.zeros((100, 64), np.float32)
for _h in range(8):
    for _w in range(8):
        _SEL0[_h * 10 + _w, _h * 8 + _w] = 1.0

_ECS = np.arange(E * 16)
_REP0 = (np.arange(E)[:, None] == _ECS[None, :] // 16).astype(np.float32)
_MSK0 = (np.arange(16)[:, None] == _ECS[None, :] % 16).astype(np.float32)
_ECS1 = np.arange(E * 32)
_REP1 = (np.arange(E)[:, None] == _ECS1[None, :] // 32).astype(np.float32)
_MSK1 = (np.arange(32)[:, None] == _ECS1[None, :] % 32).astype(np.float32)


def _shift(img, s):
    if s == 0:
        return img
    n, q = img.shape
    return jnp.concatenate([img[:, s:], jnp.zeros((n, s), img.dtype)], axis=1)


def _gates(logits):
    m = jnp.max(logits, axis=1, keepdims=True)
    e = jnp.exp(logits - m)
    p = e / jnp.sum(e, axis=1, keepdims=True)
    ecols = jax.lax.broadcasted_iota(jnp.int32, p.shape, 1)
    i1 = jnp.argmax(p, axis=1)[:, None]
    m1 = ecols == i1
    v1 = jnp.max(p, axis=1, keepdims=True)
    p2 = jnp.where(m1, -jnp.inf, p)
    i2 = jnp.argmax(p2, axis=1)[:, None]
    m2 = ecols == i2
    v2 = jnp.max(p2, axis=1, keepdims=True)
    d = v1 + v2 + 1e-6
    return m1 * (v1 / d) + m2 * (v2 / d)


def _moe_body(x_ref, wg0_ref, w0_ref, b0_ref, wg1_ref, w1_ref, b1_ref,
              sel0_ref, rep0_ref, msk0_ref, rep1_ref, msk1_ref, h1_ref):
    x = x_ref[...]
    g0 = _gates(x @ wg0_ref[...])
    G0 = jax.lax.dot(g0, rep0_ref[...])
    S0 = [_shift(x, i * 10 + j) for i in range(3) for j in range(3)]
    w0 = w0_ref[...]
    b0 = b0_ref[...]
    msk0 = msk0_ref[...]
    sel0 = sel0_ref[...]
    rows = []
    for t in range(N):
        S0t = jnp.concatenate([s[t:t + 1, :] for s in S0], axis=0)
        r = jnp.maximum(jax.lax.dot(w0, S0t) + b0, 0.0)
        m = msk0 * G0[t:t + 1, :]
        bt = jax.lax.dot(jax.lax.dot(m, r), sel0)
        rows.append(jnp.concatenate([bt[c:c + 1, :] for c in range(16)],
                                    axis=1))
    img1 = jnp.concatenate(rows, axis=0)

    g1 = _gates(img1 @ wg1_ref[...])
    G1 = jax.lax.dot(g1, rep1_ref[...])
    S1 = [_shift(img1, i * 32 + j) for i in range(3) for j in range(3)]
    w1 = w1_ref[...]
    b1 = b1_ref[...]
    msk1 = msk1_ref[...]
    for t in range(N):
        S1t = jnp.concatenate([s[t:t + 1, :] for s in S1], axis=0)
        r = jnp.maximum(jax.lax.dot(w1, S1t) + b1, 0.0)
        m = msk1 * G1[t:t + 1, :]
        h1_ref[t] = jax.lax.dot(m, r)


_NB = 16
_RK = 128 // _NB


def _fc1_body(flat_ref, w1_ref, z_ref):
    z_ref[...] = jax.lax.dot_general(
        w1_ref[...], flat_ref[...], (((1,), (1,)), ((), ())),
        preferred_element_type=jnp.float32)


def _fc2_body(z_ref, b1_ref, w2_ref, b2_ref, out_ref):
    z = jnp.maximum(z_ref[...] + b1_ref[...], 0.0)
    o = jax.lax.dot_general(
        z, w2_ref[...], (((1,), (1,)), ((), ())),
        preferred_element_type=jnp.float32)
    o = o + b2_ref[...]
    mx = jnp.max(o, axis=1, keepdims=True)
    l = o - mx
    out_ref[...] = l - jnp.log(jnp.sum(jnp.exp(l), axis=1, keepdims=True))


def kernel(x, w_gate0, conv_w0, conv_b0, w_gate1, conv_w1, conv_b1,
           fc1_w, fc1_b, fc2_w, fc2_b):
    w0 = conv_w0.reshape(E * 16, 9)
    b0 = conv_b0.reshape(E * 16, 1)
    w1 = conv_w1.reshape(E * 32, 9)
    b1 = conv_b1.reshape(E * 32, 1)

    h1_pad = pl.pallas_call(
        _moe_body,
        out_shape=jax.ShapeDtypeStruct((N, 32, 1024), jnp.float32),
    )(x, w_gate0, w0, b0, w_gate1, w1, b1, jnp.asarray(_SEL0),
      jnp.asarray(_REP0), jnp.asarray(_MSK0),
      jnp.asarray(_REP1), jnp.asarray(_MSK1))

    flat = h1_pad.reshape(N, 32, 32, 32)[:, :, :30, :30].reshape(1, 460800)

    zc = pl.pallas_call(
        _fc1_body,
        grid=(_NB,),
        in_specs=[
            pl.BlockSpec((1, 460800), lambda i: (0, 0)),
            pl.BlockSpec((_RK, 460800), lambda i: (i, 0)),
        ],
        out_specs=pl.BlockSpec((_RK, 1), lambda i: (i, 0)),
        out_shape=jax.ShapeDtypeStruct((128, 1), jnp.float32),
    )(flat, fc1_w)

    out = pl.pallas_call(
        _fc2_body,
        in_specs=[
            pl.BlockSpec((1, 128), lambda i: (0, 0)),
            pl.BlockSpec((1, 128), lambda i: (0, 0)),
            pl.BlockSpec((28800, 128), lambda i: (0, 0)),
            pl.BlockSpec((1, 28800), lambda i: (0, 0)),
        ],
        out_specs=pl.BlockSpec((1, 28800), lambda i: (0, 0)),
        out_shape=jax.ShapeDtypeStruct((1, 28800), jnp.float32),
        grid=(1,),
    )(zc.reshape(1, 128), fc1_b.reshape(1, 128), fc2_w, fc2_b.reshape(1, 28800))

    return out


# speedup vs baseline: 2.1411x; 1.0600x over previous
import jax
import jax.numpy as jnp
import numpy as np
from jax.experimental import pallas as pl
from jax.experimental.pallas import tpu as pltpu

E = 8
N = 16

_SEL0 = np.zeros((100, 64), np.float32)
for _h in range(8):
    for _w in range(8):
        _SEL0[_h * 10 + _w, _h * 8 + _w] = 1.0

_SEL1 = np.zeros((1024, 900), np.float32)
for _h in range(30):
    for _w in range(30):
        _SEL1[_h * 32 + _w, _h * 30 + _w] = 1.0

_ECS = np.arange(E * 16)
_REP0 = (np.arange(E)[:, None] == _ECS[None, :] // 16).astype(np.float32)
_MSK0 = (np.arange(16)[:, None] == _ECS[None, :] % 16).astype(np.float32)
_ECS1 = np.arange(E * 32)
_REP1 = (np.arange(E)[:, None] == _ECS1[None, :] // 32).astype(np.float32)
_MSK1 = (np.arange(32)[:, None] == _ECS1[None, :] % 32).astype(np.float32)


def _shift(img, s):
    if s == 0:
        return img
    n, q = img.shape
    return jnp.concatenate([img[:, s:], jnp.zeros((n, s), img.dtype)], axis=1)


def _gates(logits):
    m = jnp.max(logits, axis=1, keepdims=True)
    e = jnp.exp(logits - m)
    p = e / jnp.sum(e, axis=1, keepdims=True)
    ecols = jax.lax.broadcasted_iota(jnp.int32, p.shape, 1)
    i1 = jnp.argmax(p, axis=1)[:, None]
    m1 = ecols == i1
    v1 = jnp.max(p, axis=1, keepdims=True)
    p2 = jnp.where(m1, -jnp.inf, p)
    i2 = jnp.argmax(p2, axis=1)[:, None]
    m2 = ecols == i2
    v2 = jnp.max(p2, axis=1, keepdims=True)
    d = v1 + v2 + 1e-6
    return m1 * (v1 / d) + m2 * (v2 / d)


def _moe_body(x_ref, wg0_ref, w0_ref, b0_ref, wg1_ref, w1_ref, b1_ref,
              sel0_ref, rep0_ref, msk0_ref, rep1_ref, msk1_ref, sel1_ref,
              h1_ref):
    x = x_ref[...]
    g0 = _gates(x @ wg0_ref[...])
    G0 = jax.lax.dot(g0, rep0_ref[...])
    S0 = [_shift(x, i * 10 + j) for i in range(3) for j in range(3)]
    w0 = w0_ref[...]
    b0 = b0_ref[...]
    msk0 = msk0_ref[...]
    sel0 = sel0_ref[...]
    rows = []
    for t in range(N):
        S0t = jnp.concatenate([s[t:t + 1, :] for s in S0], axis=0)
        r = jnp.maximum(jax.lax.dot(w0, S0t) + b0, 0.0)
        m = msk0 * G0[t:t + 1, :]
        bt = jax.lax.dot(jax.lax.dot(m, r), sel0)
        rows.append(jnp.concatenate([bt[c:c + 1, :] for c in range(16)],
                                    axis=1))
    img1 = jnp.concatenate(rows, axis=0)

    g1 = _gates(img1 @ wg1_ref[...])
    G1 = jax.lax.dot(g1, rep1_ref[...])
    S1 = [_shift(img1, i * 32 + j) for i in range(3) for j in range(3)]
    w1 = w1_ref[...]
    b1 = b1_ref[...]
    msk1 = msk1_ref[...]
    sel1 = sel1_ref[...]
    for t in range(N):
        S1t = jnp.concatenate([s[t:t + 1, :] for s in S1], axis=0)
        r = jnp.maximum(jax.lax.dot(w1, S1t) + b1, 0.0)
        m = msk1 * G1[t:t + 1, :]
        h1_ref[t] = jax.lax.dot(jax.lax.dot(m, r), sel1)


_NB = 16
_RK = 128 // _NB


def _head_body(flat_ref, w1_ref, b1_ref, w2_ref, b2_ref, out_ref, zacc_ref):
    i = pl.program_id(0)
    zacc_ref[pl.ds(i * _RK, _RK), :] = jax.lax.dot_general(
        w1_ref[...], flat_ref[...], (((1,), (1,)), ((), ())),
        preferred_element_type=jnp.float32)

    @pl.when(i == _NB - 1)
    def _():
        z = jnp.maximum(zacc_ref[...] + b1_ref[...], 0.0)
        o = jax.lax.dot_general(
            z, w2_ref[...], (((0,), (1,)), ((), ())),
            preferred_element_type=jnp.float32)
        o = o + b2_ref[...]
        mx = jnp.max(o, axis=1, keepdims=True)
        l = o - mx
        out_ref[...] = l - jnp.log(jnp.sum(jnp.exp(l), axis=1, keepdims=True))


def kernel(x, w_gate0, conv_w0, conv_b0, w_gate1, conv_w1, conv_b1,
           fc1_w, fc1_b, fc2_w, fc2_b):
    w0 = conv_w0.reshape(E * 16, 9)
    b0 = conv_b0.reshape(E * 16, 1)
    w1 = conv_w1.reshape(E * 32, 9)
    b1 = conv_b1.reshape(E * 32, 1)

    h1 = pl.pallas_call(
        _moe_body,
        out_shape=jax.ShapeDtypeStruct((N, 32, 900), jnp.float32),
    )(x, w_gate0, w0, b0, w_gate1, w1, b1, jnp.asarray(_SEL0),
      jnp.asarray(_REP0), jnp.asarray(_MSK0),
      jnp.asarray(_REP1), jnp.asarray(_MSK1), jnp.asarray(_SEL1))

    flat = h1.reshape(1, 460800)

    out = pl.pallas_call(
        _head_body,
        grid=(_NB,),
        in_specs=[
            pl.BlockSpec((1, 460800), lambda i: (0, 0)),
            pl.BlockSpec((_RK, 460800), lambda i: (i, 0)),
            pl.BlockSpec((128, 1), lambda i: (0, 0)),
            pl.BlockSpec((28800, 128), lambda i: (0, 0)),
            pl.BlockSpec((1, 28800), lambda i: (0, 0)),
        ],
        out_specs=pl.BlockSpec((1, 28800), lambda i: (0, 0)),
        out_shape=jax.ShapeDtypeStruct((1, 28800), jnp.float32),
        scratch_shapes=[pltpu.VMEM((128, 1), jnp.float32)],
    )(flat, fc1_w, fc1_b.reshape(128, 1), fc2_w, fc2_b.reshape(1, 28800))

    return out
